# baseline (speedup 1.0000x reference)
import functools

import jax
import jax.numpy as jnp
from jax import lax
from jax.experimental import pallas as pl
from jax.experimental.pallas import tpu as pltpu
from jax.experimental.pallas import tpu_sc as plsc

_B, _S, _H, _E, _K, _D, _C = 32, 2048, 1024, 16, 4, 256, 3
_V = 30522
_VP = 30720
_KC = 1024
_NKC = _VP // _KC
_L = 16


def _hist_body(ids_hbm, table_hbm, counts_hbm, cls_hbm,
               ids_v, cnt_v, cls_v, sem):
    wid = lax.axis_index("s") * 2 + lax.axis_index("c")

    pltpu.sync_copy(ids_hbm.at[wid], ids_v)

    cls_cp = pltpu.async_copy(table_hbm.at[ids_v.at[pl.ds(0, 1)]], cls_v, sem)

    zeros = jnp.zeros((_L,), jnp.float32)

    def zbody(j, _):
        cnt_v[pl.ds(j * _L, _L)] = zeros
        return 0

    lax.fori_loop(0, _VP // _L, zbody, 0, unroll=4)

    ones = jnp.ones((_L,), jnp.float32)

    def sbody(g, _):
        idx = ids_v[pl.ds(g * _L, _L)]
        plsc.addupdate_scatter(cnt_v, [idx], ones)
        return 0

    lax.fori_loop(0, _S // _L, sbody, 0)

    pltpu.sync_copy(cnt_v, counts_hbm.at[wid])
    cls_cp.wait()
    pltpu.sync_copy(cls_v.at[0], cls_hbm.at[wid])


@functools.cache
def _hist_call():
    return pl.kernel(
        _hist_body,
        out_type=(jax.ShapeDtypeStruct((_B, _VP), jnp.float32),
                  jax.ShapeDtypeStruct((_B, _H), jnp.float32)),
        mesh=plsc.VectorSubcoreMesh(core_axis_name="c", subcore_axis_name="s",
                                    num_cores=2, num_subcores=16),
        compiler_params=pltpu.CompilerParams(needs_layout_passes=False),
        scratch_types=[
            pltpu.VMEM((_S,), jnp.int32),
            pltpu.VMEM((_VP,), jnp.float32),
            pltpu.VMEM((1, _H), jnp.float32),
            pltpu.SemaphoreType.DMA,
        ],
    )


def _tail_body(counts_ref, cls_ref, table_hbm, tail_hbm, dW, db, oW, ob, rW, rb,
               eW1, eb1, elg, elb, eW2s, eb2, pW, pb, f1W, f1b, l2g, l2b,
               f2W, f2b, out_ref, buf0, buf1, buf2, buf3, buf4,
               sem0, sem1, sem2, sem3, sem4):
    f32 = jnp.float32
    bufs = (buf0, buf1, buf2, buf3, buf4)
    sems = (sem0, sem1, sem2, sem3, sem4)
    _NB = len(bufs)

    def start(i):
        if i < _NKC - 1:
            src = table_hbm.at[pl.ds(i * _KC, _KC)]
        else:
            src = tail_hbm.at[pl.ds(0, _KC)]
        return pltpu.async_copy(src, bufs[i % _NB], sems[i % _NB])

    cps = [start(i) for i in range(_NB)]

    cls = cls_ref[...]
    x = jnp.tanh(jnp.dot(cls, dW[...], preferred_element_type=f32,
                 precision=lax.Precision.HIGHEST) + db[...])
    orig = jnp.dot(x, oW[...], preferred_element_type=f32,
                 precision=lax.Precision.HIGHEST) + ob[...]

    rl = jnp.dot(cls, rW[...], preferred_element_type=f32,
                 precision=lax.Precision.HIGHEST) + rb[...]

    iota_e = lax.broadcasted_iota(jnp.int32, (_B, _E), 1)
    neg = jnp.float32(-1e30)
    work = rl
    sel = jnp.zeros((_B, _E), jnp.bool_)
    for _ in range(_K):
        m = jnp.max(work, axis=-1, keepdims=True)
        is_max = work == m
        first = jnp.min(jnp.where(is_max, iota_e, _E), axis=-1, keepdims=True)
        pick = iota_e == first
        sel = jnp.logical_or(sel, pick)
        work = jnp.where(pick, neg, work)
    vals = jnp.where(sel, rl, neg)
    mx = jnp.max(vals, axis=-1, keepdims=True)
    ex = jnp.where(sel, jnp.exp(vals - mx), 0.0)
    w = ex / jnp.sum(ex, axis=-1, keepdims=True)

    acc = jnp.zeros((_B, _H), f32)
    for i in range(_NKC):
        cps[i % _NB].wait()
        c = counts_ref[:, i * _KC:(i + 1) * _KC]
        acc = acc + jnp.dot(c, bufs[i % _NB][...],
                            preferred_element_type=f32)
        if i + _NB < _NKC:
            cps[i % _NB] = start(i + _NB)
    mean_emb = acc * (1.0 / _S)

    gw_blocks = []
    for e in range(_E):
        h1 = jnp.dot(mean_emb, eW1[e], preferred_element_type=f32,
                 precision=lax.Precision.HIGHEST) + eb1[e]
        mu = jnp.mean(h1, axis=-1, keepdims=True)
        var = jnp.mean((h1 - mu) ** 2, axis=-1, keepdims=True)
        h1 = (h1 - mu) / jnp.sqrt(var + 1e-5) * elg[e] + elb[e]
        h1 = 0.5 * h1 * (1.0 + lax.erf(h1 * (2.0 ** -0.5)))
        gw_blocks.append(h1 * w[:, e:e + 1])
    gw = jnp.concatenate(gw_blocks, axis=1)
    u = (jnp.dot(gw, eW2s[...], preferred_element_type=f32,
                 precision=lax.Precision.HIGHEST)
         + jnp.dot(w, eb2[...], preferred_element_type=f32,
                 precision=lax.Precision.HIGHEST))
    moe = jnp.dot(u, pW[...], preferred_element_type=f32,
                 precision=lax.Precision.HIGHEST) + pb[...]

    comb = jnp.concatenate([orig, moe], axis=1)
    y = jnp.dot(comb, f1W[...], preferred_element_type=f32,
                 precision=lax.Precision.HIGHEST) + f1b[...]
    mu = jnp.mean(y, axis=-1, keepdims=True)
    var = jnp.mean((y - mu) ** 2, axis=-1, keepdims=True)
    y = (y - mu) / jnp.sqrt(var + 1e-5) * l2g[...] + l2b[...]
    y = jnp.maximum(y, 0.0)
    out_ref[...] = jnp.dot(y, f2W[...], preferred_element_type=f32,
                 precision=lax.Precision.HIGHEST) + f2b[...]


def kernel(input_ids, attention_mask, embed_table, dense_W, dense_b, out_W,
           out_b, router_W, router_b, exp_W1, exp_b1, exp_ln_g, exp_ln_b,
           exp_W2, exp_b2, proj_W, proj_b, fc1_W, fc1_b, ln2_g, ln2_b,
           fc2_W, fc2_b):
    del attention_mask
    ids = input_ids.astype(jnp.int32)
    counts, cls = _hist_call()(ids, embed_table)
    tail_pad = jnp.pad(embed_table[(_NKC - 1) * _KC:],
                       ((0, _NKC * _KC - _V), (0, 0)))
    return pl.pallas_call(
        _tail_body,
        out_shape=jax.ShapeDtypeStruct((_B, _C), jnp.float32),
        in_specs=[pl.BlockSpec(memory_space=pltpu.MemorySpace.VMEM),
                  pl.BlockSpec(memory_space=pltpu.MemorySpace.VMEM),
                  pl.BlockSpec(memory_space=pltpu.MemorySpace.HBM),
                  pl.BlockSpec(memory_space=pltpu.MemorySpace.HBM)] +
                 [pl.BlockSpec(memory_space=pltpu.MemorySpace.VMEM)] * 20,
        scratch_shapes=[pltpu.VMEM((_KC, _H), jnp.float32),
                        pltpu.VMEM((_KC, _H), jnp.float32),
                        pltpu.VMEM((_KC, _H), jnp.float32),
                        pltpu.VMEM((_KC, _H), jnp.float32),
                        pltpu.VMEM((_KC, _H), jnp.float32),
                        pltpu.SemaphoreType.DMA,
                        pltpu.SemaphoreType.DMA,
                        pltpu.SemaphoreType.DMA,
                        pltpu.SemaphoreType.DMA,
                        pltpu.SemaphoreType.DMA],
    )(counts, cls, embed_table, tail_pad, dense_W, dense_b, out_W, out_b,
      router_W, router_b, exp_W1, exp_b1, exp_ln_g, exp_ln_b,
      exp_W2.reshape(_E * _D, _D),
      exp_b2, proj_W, proj_b, fc1_W, fc1_b, ln2_g, ln2_b, fc2_W, fc2_b)


# speedup vs baseline: 1.1592x; 1.1592x over previous
import functools

import jax
import jax.numpy as jnp
from jax import lax
from jax.experimental import pallas as pl
from jax.experimental.pallas import tpu as pltpu
from jax.experimental.pallas import tpu_sc as plsc

_B, _S, _H, _E, _K, _D, _C = 32, 2048, 1024, 16, 4, 256, 3
_V = 30522
_VP = 30720
_KC = 1024
_NKC = _VP // _KC
_L = 16


def _hist_body(ids_hbm, table_hbm, counts_hbm, cls_hbm,
               ids_v, cnt_v, cls_v, sem):
    wid = lax.axis_index("s") * 2 + lax.axis_index("c")

    pltpu.sync_copy(ids_hbm.at[wid], ids_v)

    cls_cp = pltpu.async_copy(table_hbm.at[ids_v.at[pl.ds(0, 1)]], cls_v, sem)

    zeros = jnp.zeros((_L,), jnp.float32)

    def zbody(j, _):
        cnt_v[pl.ds(j * _L, _L)] = zeros
        return 0

    lax.fori_loop(0, _VP // _L, zbody, 0, unroll=4)

    ones = jnp.ones((_L,), jnp.float32)

    def sbody(g, _):
        idx = ids_v[pl.ds(g * _L, _L)]
        plsc.addupdate_scatter(cnt_v, [idx], ones)
        return 0

    lax.fori_loop(0, _S // _L, sbody, 0)

    pltpu.sync_copy(cnt_v, counts_hbm.at[wid])
    cls_cp.wait()
    pltpu.sync_copy(cls_v.at[0], cls_hbm.at[wid])


@functools.cache
def _hist_call():
    return pl.kernel(
        _hist_body,
        out_type=(jax.ShapeDtypeStruct((_B, _VP), jnp.float32),
                  jax.ShapeDtypeStruct((_B, _H), jnp.float32)),
        mesh=plsc.VectorSubcoreMesh(core_axis_name="c", subcore_axis_name="s",
                                    num_cores=2, num_subcores=16),
        compiler_params=pltpu.CompilerParams(needs_layout_passes=False),
        scratch_types=[
            pltpu.VMEM((_S,), jnp.int32),
            pltpu.VMEM((_VP,), jnp.float32),
            pltpu.VMEM((1, _H), jnp.float32),
            pltpu.SemaphoreType.DMA,
        ],
    )


def _tail_body(counts_ref, cls_ref, table_hbm, tail_hbm, dW, db, oW, ob, rW, rb,
               eW1, eb1, elg, elb, eW2s, eb2, pW, pb, f1W, f1b, l2g, l2b,
               f2W, f2b, out_ref, buf0, buf1, buf2, buf3, buf4,
               sem0, sem1, sem2, sem3, sem4):
    f32 = jnp.float32
    bufs = (buf0, buf1, buf2, buf3, buf4)
    sems = (sem0, sem1, sem2, sem3, sem4)
    _NB = len(bufs)

    def start(i):
        if i < _NKC - 1:
            src = table_hbm.at[pl.ds(i * _KC, _KC)]
        else:
            src = tail_hbm.at[pl.ds(0, _KC)]
        return pltpu.async_copy(src, bufs[i % _NB], sems[i % _NB])

    cps = [start(i) for i in range(_NB)]

    cls = cls_ref[...]
    x = jnp.tanh(jnp.dot(cls, dW[...], preferred_element_type=f32) + db[...])
    orig = jnp.dot(x, oW[...], preferred_element_type=f32) + ob[...]

    rl = jnp.dot(cls, rW[...], preferred_element_type=f32) + rb[...]

    iota_e = lax.broadcasted_iota(jnp.int32, (_B, _E), 1)
    neg = jnp.float32(-1e30)
    work = rl
    sel = jnp.zeros((_B, _E), jnp.bool_)
    for _ in range(_K):
        m = jnp.max(work, axis=-1, keepdims=True)
        is_max = work == m
        first = jnp.min(jnp.where(is_max, iota_e, _E), axis=-1, keepdims=True)
        pick = iota_e == first
        sel = jnp.logical_or(sel, pick)
        work = jnp.where(pick, neg, work)
    vals = jnp.where(sel, rl, neg)
    mx = jnp.max(vals, axis=-1, keepdims=True)
    ex = jnp.where(sel, jnp.exp(vals - mx), 0.0)
    w = ex / jnp.sum(ex, axis=-1, keepdims=True)

    acc = jnp.zeros((_B, _H), f32)
    for i in range(_NKC):
        cps[i % _NB].wait()
        c = counts_ref[:, i * _KC:(i + 1) * _KC]
        acc = acc + jnp.dot(c, bufs[i % _NB][...],
                            preferred_element_type=f32)
        if i + _NB < _NKC:
            cps[i % _NB] = start(i + _NB)
    mean_emb = acc * (1.0 / _S)

    gw_blocks = []
    for e in range(_E):
        h1 = jnp.dot(mean_emb, eW1[e], preferred_element_type=f32) + eb1[e]
        mu = jnp.mean(h1, axis=-1, keepdims=True)
        var = jnp.mean((h1 - mu) ** 2, axis=-1, keepdims=True)
        h1 = (h1 - mu) / jnp.sqrt(var + 1e-5) * elg[e] + elb[e]
        h1 = 0.5 * h1 * (1.0 + lax.erf(h1 * (2.0 ** -0.5)))
        gw_blocks.append(h1 * w[:, e:e + 1])
    gw = jnp.concatenate(gw_blocks, axis=1)
    u = (jnp.dot(gw, eW2s[...], preferred_element_type=f32)
         + jnp.dot(w, eb2[...], preferred_element_type=f32))
    moe = jnp.dot(u, pW[...], preferred_element_type=f32) + pb[...]

    comb = jnp.concatenate([orig, moe], axis=1)
    y = jnp.dot(comb, f1W[...], preferred_element_type=f32) + f1b[...]
    mu = jnp.mean(y, axis=-1, keepdims=True)
    var = jnp.mean((y - mu) ** 2, axis=-1, keepdims=True)
    y = (y - mu) / jnp.sqrt(var + 1e-5) * l2g[...] + l2b[...]
    y = jnp.maximum(y, 0.0)
    out_ref[...] = jnp.dot(y, f2W[...], preferred_element_type=f32) + f2b[...]


def kernel(input_ids, attention_mask, embed_table, dense_W, dense_b, out_W,
           out_b, router_W, router_b, exp_W1, exp_b1, exp_ln_g, exp_ln_b,
           exp_W2, exp_b2, proj_W, proj_b, fc1_W, fc1_b, ln2_g, ln2_b,
           fc2_W, fc2_b):
    del attention_mask
    ids = input_ids.astype(jnp.int32)
    counts, cls = _hist_call()(ids, embed_table)
    tail_pad = jnp.pad(embed_table[(_NKC - 1) * _KC:],
                       ((0, _NKC * _KC - _V), (0, 0)))
    return pl.pallas_call(
        _tail_body,
        out_shape=jax.ShapeDtypeStruct((_B, _C), jnp.float32),
        in_specs=[pl.BlockSpec(memory_space=pltpu.MemorySpace.VMEM),
                  pl.BlockSpec(memory_space=pltpu.MemorySpace.VMEM),
                  pl.BlockSpec(memory_space=pltpu.MemorySpace.HBM),
                  pl.BlockSpec(memory_space=pltpu.MemorySpace.HBM)] +
                 [pl.BlockSpec(memory_space=pltpu.MemorySpace.VMEM)] * 20,
        scratch_shapes=[pltpu.VMEM((_KC, _H), jnp.float32),
                        pltpu.VMEM((_KC, _H), jnp.float32),
                        pltpu.VMEM((_KC, _H), jnp.float32),
                        pltpu.VMEM((_KC, _H), jnp.float32),
                        pltpu.VMEM((_KC, _H), jnp.float32),
                        pltpu.SemaphoreType.DMA,
                        pltpu.SemaphoreType.DMA,
                        pltpu.SemaphoreType.DMA,
                        pltpu.SemaphoreType.DMA,
                        pltpu.SemaphoreType.DMA],
    )(counts, cls, embed_table, tail_pad, dense_W, dense_b, out_W, out_b,
      router_W, router_b, exp_W1, exp_b1, exp_ln_g, exp_ln_b,
      exp_W2.reshape(_E * _D, _D),
      exp_b2, proj_W, proj_b, fc1_W, fc1_b, ln2_g, ln2_b, fc2_W, fc2_b)
